# Initial kernel scaffold; baseline (speedup 1.0000x reference)
import jax
import jax.numpy as jnp
from jax.experimental import pallas as pl


def kernel(h, src_idx, dst_idx):
    raise NotImplementedError("write your pallas kernel here")


# speedup vs baseline: 3.2155x; 3.2155x over previous
import functools

import jax
import jax.numpy as jnp
from jax import lax
from jax.experimental import pallas as pl
from jax.experimental.pallas import tpu as pltpu
from jax.experimental.pallas import tpu_sc as plsc

_NC, _NS, _L = 2, 16, 16
_NW = _NC * _NS

_E = 320000
_D = 128
_PER_W = _E // _NW
_CHUNK = 400
_NCHUNK = _PER_W // _CHUNK


def _dot_link_sc(h, src_idx, dst_idx):
    mesh = plsc.VectorSubcoreMesh(core_axis_name="c", subcore_axis_name="s")

    @functools.partial(
        pl.kernel,
        mesh=mesh,
        compiler_params=pltpu.CompilerParams(needs_layout_passes=False),
        out_type=jax.ShapeDtypeStruct((_E,), jnp.float32),
        scratch_types=[
            pltpu.VMEM((_CHUNK,), jnp.int32),
            pltpu.VMEM((_CHUNK,), jnp.int32),
            pltpu.VMEM((_CHUNK, _D), jnp.float32),
            pltpu.VMEM((_CHUNK, _D), jnp.float32),
            pltpu.VMEM((_CHUNK,), jnp.float32),
            pltpu.SemaphoreType.DMA,
            pltpu.SemaphoreType.DMA,
        ],
    )
    def k(h_hbm, sidx_hbm, didx_hbm, out_hbm,
          sidx_v, didx_v, srows_v, drows_v, out_v, sem_a, sem_b):
        wid = lax.axis_index("s") * _NC + lax.axis_index("c")
        base = wid * _PER_W

        def chunk_body(c, carry):
            off = base + c * _CHUNK
            pltpu.sync_copy(sidx_hbm.at[pl.ds(off, _CHUNK)], sidx_v)
            pltpu.sync_copy(didx_hbm.at[pl.ds(off, _CHUNK)], didx_v)
            cp_s = pltpu.async_copy(h_hbm.at[sidx_v], srows_v, sem_a)
            cp_d = pltpu.async_copy(h_hbm.at[didx_v], drows_v, sem_b)
            cp_s.wait()
            cp_d.wait()

            lane = lax.iota(jnp.int32, _L)

            def group_body(g, c2):
                acc = jnp.zeros((_L,), jnp.float32)
                for k in range(_L):
                    e = g * _L + k
                    s = srows_v[e, pl.ds(0, _L)] * drows_v[e, pl.ds(0, _L)]
                    for j in range(1, _D // _L):
                        s = s + (srows_v[e, pl.ds(j * _L, _L)]
                                 * drows_v[e, pl.ds(j * _L, _L)])
                    acc = jnp.where(lane == k, jnp.sum(s), acc)
                out_v[pl.ds(g * _L, _L)] = acc
                return c2

            lax.fori_loop(0, _CHUNK // _L, group_body, 0)
            pltpu.sync_copy(out_v, out_hbm.at[pl.ds(off, _CHUNK)])
            return carry

        lax.fori_loop(0, _NCHUNK, chunk_body, 0)

    return k(h, src_idx, dst_idx)


def kernel(h, src_idx, dst_idx):
    return _dot_link_sc(h, src_idx.astype(jnp.int32), dst_idx.astype(jnp.int32))
